# baseline (speedup 1.0000x reference)
import functools

import jax
import jax.numpy as jnp
from jax import lax
from jax.experimental import pallas as pl
from jax.experimental.pallas import tpu as pltpu
from jax.experimental.pallas import tpu_sc as plsc

N_FRAMES = 4096
K_TAB = 16
D_ROW = 18 * 512


import numpy as np

_I32_MIN = np.int32(-2147483648)
_RANKS = (1023, 1024, 3071, 3072)


def _key_to_float(v):
  b = jnp.where(v >= 0, v, _I32_MIN - v)
  return lax.bitcast_convert_type(b, jnp.float32)


def _index_kernel(pitch_ref, idx_ref):
  x = pitch_ref[...]
  b = lax.bitcast_convert_type(x, jnp.int32)
  k = jnp.where(b >= 0, b, _I32_MIN - b)

  stats = []
  for r in _RANKS:
    r = jnp.int32(r)
    cnt0 = jnp.sum((k < 0).astype(jnp.int32))
    ans = jnp.where(cnt0 <= r, jnp.int32(0), _I32_MIN)
    for bit in range(30, -1, -1):
      t = ans + jnp.int32(1 << bit)
      cnt = jnp.sum((k < t).astype(jnp.int32))
      ans = jnp.where(cnt <= r, t, ans)
    stats.append(_key_to_float(ans))
  s1023, s1024, s3071, s3072 = stats

  low = s1023 * jnp.float32(0.25) + s1024 * jnp.float32(0.75)
  high = s3071 * jnp.float32(0.75) + s3072 * jnp.float32(0.25)

  pt = x - low
  pt = pt / high
  pt = pt * jnp.float32(16.0)
  m = pt - jnp.float32(16.0) * jnp.floor(pt * jnp.float32(0.0625))
  idx = jnp.round(m).astype(jnp.int32) % K_TAB
  idx_ref[...] = idx


def _compute_indices(pitch_track):
  pitch2d = pitch_track.reshape(32, 128)
  idx2d = pl.pallas_call(
      _index_kernel,
      out_shape=jax.ShapeDtypeStruct((32, 128), jnp.int32),
  )(pitch2d)
  return idx2d.reshape(-1)


NC, NS = 2, 16
NW = NC * NS
FRAMES_PER_W = N_FRAMES // NW
CHUNK = 4
NCHUNK = FRAMES_PER_W // CHUNK


def _make_gather():
  mesh = plsc.VectorSubcoreMesh(core_axis_name="c", subcore_axis_name="s")

  @functools.partial(
      pl.kernel,
      out_type=jax.ShapeDtypeStruct((N_FRAMES, D_ROW), jnp.float32),
      mesh=mesh,
      scratch_types=[
          pltpu.VMEM((NCHUNK, CHUNK), jnp.int32),
          pltpu.VMEM((CHUNK, D_ROW), jnp.float32),
          pltpu.VMEM((CHUNK, D_ROW), jnp.float32),
          pltpu.SemaphoreType.DMA,
          pltpu.SemaphoreType.DMA,
      ],
  )
  def gather_k(table_hbm, idx_hbm, out_hbm, idx_v, buf0, buf1, gs0, gs1):
    wid = lax.axis_index("s") * NC + lax.axis_index("c")
    base = wid * FRAMES_PER_W
    pltpu.sync_copy(idx_hbm.at[pl.ds(wid * NCHUNK, NCHUNK)], idx_v)
    bufs, gsems = (buf0, buf1), (gs0, gs1)

    def g_copy(g, b):
      return pltpu.make_async_copy(table_hbm.at[idx_v.at[g]], bufs[b], gsems[b])

    def put_out(g, b):
      pltpu.sync_copy(bufs[b], out_hbm.at[pl.ds(base + g * CHUNK, CHUNK)])

    g_copy(0, 0).start()
    g_copy(1, 1).start()

    @pl.loop(0, NCHUNK - 2, step=2)
    def _main(g0):
      for b in range(2):
        g = g0 + b
        g_copy(g, b).wait()
        put_out(g, b)
        g_copy(g + 2, b).start()

    for b in range(2):
      g = NCHUNK - 2 + b
      g_copy(g, b).wait()
      put_out(g, b)

  return gather_k


_gather_cache = []


def kernel(pitch_track, latent_selection):
  if not _gather_cache:
    _gather_cache.append(_make_gather())
  table = latent_selection.reshape(K_TAB, D_ROW)
  idx = _compute_indices(pitch_track).reshape(NW * NCHUNK, CHUNK)
  out = _gather_cache[0](table, idx)
  return out.reshape(N_FRAMES, 18, 512)


# speedup vs baseline: 1.3963x; 1.3963x over previous
import functools

import jax
import jax.numpy as jnp
from jax import lax
from jax.experimental import pallas as pl
from jax.experimental.pallas import tpu as pltpu
from jax.experimental.pallas import tpu_sc as plsc

N_FRAMES = 4096
K_TAB = 16
D_ROW = 18 * 512


import numpy as np

_I32_MIN = np.int32(-2147483648)
_RANKS = (1023, 1024, 3071, 3072)


def _key_to_float(v):
  b = jnp.where(v >= 0, v, _I32_MIN - v)
  return lax.bitcast_convert_type(b, jnp.float32)


def _index_kernel(pitch_ref, idx_ref):
  x = pitch_ref[...]
  b = lax.bitcast_convert_type(x, jnp.int32)
  k = jnp.where(b >= 0, b, _I32_MIN - b)

  stats = []
  for r in _RANKS:
    r = jnp.int32(r)
    cnt0 = jnp.sum((k < 0).astype(jnp.int32))
    ans = jnp.where(cnt0 <= r, jnp.int32(0), _I32_MIN)
    for bit in range(30, -1, -1):
      t = ans + jnp.int32(1 << bit)
      cnt = jnp.sum((k < t).astype(jnp.int32))
      ans = jnp.where(cnt <= r, t, ans)
    stats.append(_key_to_float(ans))
  s1023, s1024, s3071, s3072 = stats

  low = s1023 * jnp.float32(0.25) + s1024 * jnp.float32(0.75)
  high = s3071 * jnp.float32(0.75) + s3072 * jnp.float32(0.25)

  pt = x - low
  pt = pt / high
  pt = pt * jnp.float32(16.0)
  m = pt - jnp.float32(16.0) * jnp.floor(pt * jnp.float32(0.0625))
  idx = jnp.round(m).astype(jnp.int32) % K_TAB
  idx_ref[...] = idx


def _compute_indices(pitch_track):
  pitch2d = pitch_track.reshape(32, 128)
  idx2d = pl.pallas_call(
      _index_kernel,
      out_shape=jax.ShapeDtypeStruct((32, 128), jnp.int32),
  )(pitch2d)
  return idx2d.reshape(-1)


NC, NS = 2, 16
NW = NC * NS
FRAMES_PER_W = N_FRAMES // NW
CHUNK = 4
NCHUNK = FRAMES_PER_W // CHUNK


def _make_gather():
  mesh = plsc.VectorSubcoreMesh(core_axis_name="c", subcore_axis_name="s")

  @functools.partial(
      pl.kernel,
      out_type=jax.ShapeDtypeStruct((N_FRAMES, 24, 512), jnp.float32),
      mesh=mesh,
      compiler_params=pltpu.CompilerParams(use_tc_tiling_on_sc=True),
      scratch_types=[
          pltpu.VMEM((NCHUNK, CHUNK), jnp.int32),
          pltpu.VMEM((CHUNK, 24, 512), jnp.float32),
          pltpu.VMEM((CHUNK, 24, 512), jnp.float32),
          pltpu.SemaphoreType.DMA,
          pltpu.SemaphoreType.DMA,
      ],
  )
  def gather_k(table_hbm, idx_hbm, out_hbm, idx_v, buf0, buf1, gs0, gs1):
    wid = lax.axis_index("s") * NC + lax.axis_index("c")
    base = wid * FRAMES_PER_W
    pltpu.sync_copy(idx_hbm.at[pl.ds(wid * NCHUNK, NCHUNK)], idx_v)
    bufs, gsems = (buf0, buf1), (gs0, gs1)

    def g_copy(g, b):
      return pltpu.make_async_copy(table_hbm.at[idx_v.at[g]], bufs[b], gsems[b])

    def put_out(g, b):
      pltpu.sync_copy(bufs[b], out_hbm.at[pl.ds(base + g * CHUNK, CHUNK)])

    g_copy(0, 0).start()
    g_copy(1, 1).start()

    @pl.loop(0, NCHUNK - 2, step=2)
    def _main(g0):
      for b in range(2):
        g = g0 + b
        g_copy(g, b).wait()
        put_out(g, b)
        g_copy(g + 2, b).start()

    for b in range(2):
      g = NCHUNK - 2 + b
      g_copy(g, b).wait()
      put_out(g, b)

  return gather_k


_gather_cache = []


def kernel(pitch_track, latent_selection):
  if not _gather_cache:
    _gather_cache.append(_make_gather())
  idx = _compute_indices(pitch_track).reshape(NW * NCHUNK, CHUNK)
  table_p = jnp.pad(latent_selection, ((0, 0), (0, 6), (0, 0)))
  out_p = _gather_cache[0](table_p, idx)
  return out_p[:, :18, :]


# speedup vs baseline: 1.4158x; 1.0140x over previous
import functools

import jax
import jax.numpy as jnp
from jax import lax
from jax.experimental import pallas as pl
from jax.experimental.pallas import tpu as pltpu
from jax.experimental.pallas import tpu_sc as plsc

N_FRAMES = 4096
K_TAB = 16
D_ROW = 18 * 512


import numpy as np

_I32_MIN = np.int32(-2147483648)
_RANKS = (1023, 1024, 3071, 3072)


def _key_to_float(v):
  b = jnp.where(v >= 0, v, _I32_MIN - v)
  return lax.bitcast_convert_type(b, jnp.float32)


def _index_kernel(pitch_ref, idx_ref):
  x = pitch_ref[...]
  b = lax.bitcast_convert_type(x, jnp.int32)
  k = jnp.where(b >= 0, b, _I32_MIN - b)

  stats = []
  for r in _RANKS:
    r = jnp.int32(r)
    cnt0 = jnp.sum((k < 0).astype(jnp.int32))
    ans = jnp.where(cnt0 <= r, jnp.int32(0), _I32_MIN)
    for bit in range(30, -1, -1):
      t = ans + jnp.int32(1 << bit)
      cnt = jnp.sum((k < t).astype(jnp.int32))
      ans = jnp.where(cnt <= r, t, ans)
    stats.append(_key_to_float(ans))
  s1023, s1024, s3071, s3072 = stats

  low = s1023 * jnp.float32(0.25) + s1024 * jnp.float32(0.75)
  high = s3071 * jnp.float32(0.75) + s3072 * jnp.float32(0.25)

  pt = x - low
  pt = pt / high
  pt = pt * jnp.float32(16.0)
  m = pt - jnp.float32(16.0) * jnp.floor(pt * jnp.float32(0.0625))
  idx = jnp.round(m).astype(jnp.int32) % K_TAB
  idx_ref[...] = idx


def _compute_indices(pitch_track):
  pitch2d = pitch_track.reshape(32, 128)
  idx2d = pl.pallas_call(
      _index_kernel,
      out_shape=jax.ShapeDtypeStruct((32, 128), jnp.int32),
  )(pitch2d)
  return idx2d.reshape(-1)


NC, NS = 2, 16
NW = NC * NS
FRAMES_PER_W = N_FRAMES // NW
CHUNK = 2
NCHUNK = FRAMES_PER_W // CHUNK
NBUF = 4


def _make_gather():
  mesh = plsc.VectorSubcoreMesh(core_axis_name="c", subcore_axis_name="s")

  @functools.partial(
      pl.kernel,
      out_type=jax.ShapeDtypeStruct((N_FRAMES, 24, 512), jnp.float32),
      mesh=mesh,
      compiler_params=pltpu.CompilerParams(use_tc_tiling_on_sc=True),
      scratch_types=(
          [pltpu.VMEM((NCHUNK, CHUNK), jnp.int32)]
          + [pltpu.VMEM((CHUNK, 24, 512), jnp.float32)] * NBUF
          + [pltpu.SemaphoreType.DMA] * NBUF
      ),
  )
  def gather_k(table_hbm, idx_hbm, out_hbm, idx_v, *rest):
    bufs, gsems = rest[:NBUF], rest[NBUF:]
    wid = lax.axis_index("s") * NC + lax.axis_index("c")
    base = wid * FRAMES_PER_W
    pltpu.sync_copy(idx_hbm.at[pl.ds(wid * NCHUNK, NCHUNK)], idx_v)

    def g_copy(g, b):
      return pltpu.make_async_copy(table_hbm.at[idx_v.at[g]], bufs[b], gsems[b])

    def put_out(g, b):
      pltpu.sync_copy(bufs[b], out_hbm.at[pl.ds(base + g * CHUNK, CHUNK)])

    for b in range(NBUF):
      g_copy(b, b).start()

    @pl.loop(0, NCHUNK - NBUF, step=NBUF)
    def _main(g0):
      for b in range(NBUF):
        g = g0 + b
        g_copy(g, b).wait()
        put_out(g, b)
        g_copy(g + NBUF, b).start()

    for b in range(NBUF):
      g = NCHUNK - NBUF + b
      g_copy(g, b).wait()
      put_out(g, b)

  return gather_k


_gather_cache = []


def kernel(pitch_track, latent_selection):
  if not _gather_cache:
    _gather_cache.append(_make_gather())
  idx = _compute_indices(pitch_track).reshape(NW * NCHUNK, CHUNK)
  table_p = jnp.pad(latent_selection, ((0, 0), (0, 6), (0, 0)))
  out_p = _gather_cache[0](table_p, idx)
  return out_p[:, :18, :]
